# baseline (speedup 1.0000x reference)
import jax
import jax.numpy as jnp
from jax.experimental import pallas as pl


def _layernorm(x, g, b, eps=1e-5):
    m = jnp.mean(x, axis=-1, keepdims=True)
    v = jnp.var(x, axis=-1, keepdims=True)
    return (x - m) / jnp.sqrt(v + eps) * g + b


def _gat_conv(x, src, dst, W, a_src, a_dst, bias, heads, out_ch):
    n = x.shape[0]
    xw = x @ W
    outs = []
    for h in range(heads):
        xh = xw[:, h * out_ch:(h + 1) * out_ch]
        s_src = xh @ a_src[h]
        s_dst = xh @ a_dst[h]
        e = s_src[src] + s_dst[dst]
        e = jnp.where(e > 0, e, 0.2 * e)
        emax = jax.ops.segment_max(e, dst, num_segments=n)
        ee = jnp.exp(e - emax[dst])
        denom = jax.ops.segment_sum(ee, dst, num_segments=n)
        alpha = ee / (denom[dst] + 1e-16)
        msg = xh[src] * alpha[:, None]
        outs.append(jax.ops.segment_sum(msg, dst, num_segments=n))
    return jnp.concatenate(outs, axis=-1) + bias


def _final_matmul_kernel(x_ref, xe_ref, o_ref):
    o_ref[...] = jax.lax.dot_general(
        x_ref[...], xe_ref[...], (((1,), (1,)), ((), ())),
        preferred_element_type=jnp.float32)


def _final_matmul(x, xe):
    B, K = x.shape
    N = xe.shape[0]
    blk = 256
    return pl.pallas_call(
        _final_matmul_kernel,
        grid=(B // blk,),
        in_specs=[
            pl.BlockSpec((blk, K), lambda j: (j, 0)),
            pl.BlockSpec((N, K), lambda j: (0, 0)),
        ],
        out_specs=pl.BlockSpec((blk, N), lambda j: (j, 0)),
        out_shape=jax.ShapeDtypeStruct((B, N), jnp.float32),
    )(x, xe)


def kernel(index, x_e, edge_index_e, weight, df1, df2, fc_p1_W, fc_p1_b,
           fc1_W, fc1_b, fc2_W, fc2_b, n1_g, n1_b, ne1_g, ne1_b, ne2_g,
           ne2_b, ne3_g, ne3_b, W1, a_src1, a_dst1, b1, W2, a_src2, a_dst2,
           b2, W3, a_src3, a_dst3, b3, fcg4_W, fcg4_b):
    n = x_e.shape[0]
    loops = jnp.arange(n)
    src = jnp.concatenate([edge_index_e[0], loops])
    dst = jnp.concatenate([edge_index_e[1], loops])

    w = jax.nn.sigmoid(weight)
    drug_feature = w * df1 + (1.0 - w) * df2
    x = drug_feature[index]
    x = jax.nn.relu(x @ fc_p1_W + fc_p1_b)
    x = _layernorm(x, n1_g, n1_b)
    x = jax.nn.relu(x @ fc1_W + fc1_b)
    x = x @ fc2_W + fc2_b

    xe = _layernorm(x_e, ne1_g, ne1_b)
    xe = jax.nn.relu(_gat_conv(xe, src, dst, W1, a_src1, a_dst1, b1, 10, 128))
    xe = _layernorm(xe, ne2_g, ne2_b)
    xe = jax.nn.relu(_gat_conv(xe, src, dst, W2, a_src2, a_dst2, b2, 10, 200))
    xe = _layernorm(xe, ne3_g, ne3_b)
    xe = _gat_conv(xe, src, dst, W3, a_src3, a_dst3, b3, 1, 200)
    xe = xe @ fcg4_W + fcg4_b

    xc = _final_matmul(x, xe)
    return (xc, x, xe)


# speedup vs baseline: 11.5916x; 11.5916x over previous
import functools

import jax
import jax.numpy as jnp
from jax import lax
from jax.experimental import pallas as pl
from jax.experimental.pallas import tpu as pltpu
from jax.experimental.pallas import tpu_sc as plsc

N = 10000
NP = 10240
NZ = 10000
E = 170000
EPAD = 172032
EPT = 10752
NJ = EPT // 128
B = 1024
F32 = jnp.float32
I32 = jnp.int32


def _mix_kernel(w_ref, d1_ref, d2_ref, o_ref):
    w = w_ref[0, 0]
    o_ref[...] = w * d1_ref[...] + (1.0 - w) * d2_ref[...]


def _drug_mix(weight, df1, df2):
    w = jax.nn.sigmoid(weight).reshape(1, 1)
    return pl.pallas_call(
        _mix_kernel,
        in_specs=[
            pl.BlockSpec(memory_space=pltpu.SMEM),
            pl.BlockSpec(df1.shape, lambda: (0, 0)),
            pl.BlockSpec(df2.shape, lambda: (0, 0)),
        ],
        out_specs=pl.BlockSpec(df1.shape, lambda: (0, 0)),
        out_shape=jax.ShapeDtypeStruct(df1.shape, F32),
    )(w, df1, df2)


def _ln(x, g, b, eps=1e-5):
    m = jnp.mean(x, axis=-1, keepdims=True)
    v = jnp.mean((x - m) * (x - m), axis=-1, keepdims=True)
    return (x - m) * jax.lax.rsqrt(v + eps) * g + b


def _mlp_kernel(xd_ref, w1_ref, b1_ref, g_ref, bb_ref, w2_ref, b2_ref,
                w3_ref, b3_ref, o_ref):
    x = xd_ref[...]
    x = jax.nn.relu(jax.lax.dot_general(x, w1_ref[...], (((1,), (0,)), ((), ())),
                                        preferred_element_type=F32) + b1_ref[...])
    x = _ln(x, g_ref[...], bb_ref[...])
    x = jax.nn.relu(jax.lax.dot_general(x, w2_ref[...], (((1,), (0,)), ((), ())),
                                        preferred_element_type=F32) + b2_ref[...])
    x = jax.lax.dot_general(x, w3_ref[...], (((1,), (0,)), ((), ())),
                            preferred_element_type=F32) + b3_ref[...]
    o_ref[...] = x


def _drug_mlp(xd, fc_p1_W, fc_p1_b, n1_g, n1_b, fc1_W, fc1_b, fc2_W, fc2_b):
    full = lambda a: pl.BlockSpec(a.shape, lambda: tuple(0 for _ in a.shape))
    args = (xd, fc_p1_W, fc_p1_b.reshape(1, -1), n1_g.reshape(1, -1),
            n1_b.reshape(1, -1), fc1_W, fc1_b.reshape(1, -1), fc2_W,
            fc2_b.reshape(1, -1))
    return pl.pallas_call(
        _mlp_kernel,
        in_specs=[full(a) for a in args],
        out_specs=pl.BlockSpec((B, 200), lambda: (0, 0)),
        out_shape=jax.ShapeDtypeStruct((B, 200), F32),
    )(*args)


def _gatA_kernel(xin_ref, lg_ref, lb_ref, w_ref, asrc_ref, adst_ref,
                 xw_ref, ss_ref, sd_ref):
    xn = _ln(xin_ref[...], lg_ref[...], lb_ref[...])
    xw = jax.lax.dot_general(xn, w_ref[...], (((1,), (0,)), ((), ())),
                             preferred_element_type=F32)
    xw_ref[...] = xw
    ss = jax.lax.dot_general(xw, asrc_ref[...], (((1,), (0,)), ((), ())),
                             preferred_element_type=F32)
    sd = jax.lax.dot_general(xw, adst_ref[...], (((1,), (0,)), ((), ())),
                             preferred_element_type=F32)
    ss_ref[...] = ss.T
    sd_ref[...] = sd.T


def _gatA(xin, lg, lb, W, AsrcM, AdstM, H):
    Fin = xin.shape[1]
    F = W.shape[1]
    bn = 1024
    grid = (NP // bn,)
    return pl.pallas_call(
        _gatA_kernel,
        grid=grid,
        in_specs=[
            pl.BlockSpec((bn, Fin), lambda r: (r, 0)),
            pl.BlockSpec((1, Fin), lambda r: (0, 0)),
            pl.BlockSpec((1, Fin), lambda r: (0, 0)),
            pl.BlockSpec((Fin, F), lambda r: (0, 0)),
            pl.BlockSpec((F, H), lambda r: (0, 0)),
            pl.BlockSpec((F, H), lambda r: (0, 0)),
        ],
        out_specs=[
            pl.BlockSpec((bn, F), lambda r: (r, 0)),
            pl.BlockSpec((H, bn), lambda r: (0, r)),
            pl.BlockSpec((H, bn), lambda r: (0, r)),
        ],
        out_shape=[
            jax.ShapeDtypeStruct((NP, F), F32),
            jax.ShapeDtypeStruct((H, NP), F32),
            jax.ShapeDtypeStruct((H, NP), F32),
        ],
    )(xin, lg.reshape(1, -1), lb.reshape(1, -1), W, AsrcM, AdstM)


def _gatB_kernel(nch, bn2, xw_ref, ss_ref, o_ref):
    rb = pl.program_id(1)
    rows = rb * bn2 + lax.broadcasted_iota(I32, (bn2, 1), 0)
    rmask = rows < NZ
    sh = ss_ref[0, 0].reshape(bn2, 1)
    fpos = jnp.where(rmask, jnp.exp(sh), 0.0)
    fneg = jnp.where(rmask, jnp.exp(0.2 * sh), 0.0)
    for ci in range(nch):
        xwc = xw_ref[:, 0, ci, :]
        o_ref[2 * ci] = fpos * xwc
        o_ref[2 * ci + 1] = fneg * xwc


def _gatB(xw4, ssT3, H, nch, cw):
    bn2 = 640
    hcn = H * nch
    hc2n = hcn * 2
    kern = functools.partial(_gatB_kernel, nch, bn2)
    return pl.pallas_call(
        kern,
        grid=(H, NP // bn2),
        in_specs=[
            pl.BlockSpec((bn2, 1, nch, cw), lambda h, rb: (rb, h, 0, 0)),
            pl.BlockSpec((1, 1, bn2), lambda h, rb: (h, 0, rb)),
        ],
        out_specs=pl.BlockSpec((2 * nch, bn2, cw), lambda h, rb: (h, rb, 0)),
        out_shape=jax.ShapeDtypeStruct((hc2n, NP, cw), F32),
    )(xw4, ssT3)


def _gatC_kernel(nch, bn3, relu_flag, acc_ref, den_ref, sd_ref, bias_ref,
                 o_ref):
    sd = sd_ref[0, 0].reshape(bn3, 1)
    fp = jnp.exp(sd)
    fn = jnp.exp(0.2 * sd)
    denp = den_ref[0, 0].reshape(bn3, 1)
    denn = den_ref[0, 1].reshape(bn3, 1)
    den = fp * denp + fn * denn + 1e-16
    for ci in range(nch):
        num = fp * acc_ref[2 * ci] + fn * acc_ref[2 * ci + 1]
        o = num / den + bias_ref[0, ci]
        if relu_flag:
            o = jax.nn.relu(o)
        o_ref[:, 0, ci, :] = o


def _gatC(ACC, DEN, sdT, bias, H, nch, cw, relu_flag):
    bn3 = 2560
    F = H * nch * cw
    hcn = H * nch
    kern = functools.partial(_gatC_kernel, nch, bn3, relu_flag)
    acc3 = ACC.reshape(hcn * 2, NP, cw)
    den4 = DEN.reshape(H, 2, NP)
    sd3 = sdT.reshape(H, 1, NP)
    nrb = NP // bn3
    out4 = pl.pallas_call(
        kern,
        grid=(H, nrb),
        in_specs=[
            pl.BlockSpec((2 * nch, bn3, cw), lambda h, rb: (h, rb, 0)),
            pl.BlockSpec((1, 2, bn3), lambda h, rb: (h, 0, rb)),
            pl.BlockSpec((1, 1, bn3), lambda h, rb: (h, 0, rb)),
            pl.BlockSpec((1, nch, cw), lambda h, rb: (h, 0, 0)),
        ],
        out_specs=pl.BlockSpec((bn3, 1, nch, cw), lambda h, rb: (rb, h, 0, 0)),
        out_shape=jax.ShapeDtypeStruct((NP, H, nch, cw), F32),
    )(acc3, den4, sd3, bias.reshape(H, nch, cw))
    return out4.reshape(NP, F)


def _linear_kernel(x_ref, w_ref, b_ref, o_ref):
    o_ref[...] = jax.lax.dot_general(
        x_ref[...], w_ref[...], (((1,), (0,)), ((), ())),
        preferred_element_type=F32) + b_ref[...]


def _linear(x, Wm, bv):
    bn = 2560
    K = x.shape[1]
    M = Wm.shape[1]
    return pl.pallas_call(
        _linear_kernel,
        grid=(x.shape[0] // bn,),
        in_specs=[
            pl.BlockSpec((bn, K), lambda r: (r, 0)),
            pl.BlockSpec((K, M), lambda r: (0, 0)),
            pl.BlockSpec((1, M), lambda r: (0, 0)),
        ],
        out_specs=pl.BlockSpec((bn, M), lambda r: (r, 0)),
        out_shape=jax.ShapeDtypeStruct((x.shape[0], M), F32),
    )(x, Wm, bv.reshape(1, -1))


def _final_matmul_kernel(x_ref, xe_ref, o_ref):
    o_ref[...] = jax.lax.dot_general(
        x_ref[...], xe_ref[...], (((1,), (1,)), ((), ())),
        preferred_element_type=F32)


def _final_matmul(x, xe):
    Bm, K = x.shape
    Nm = xe.shape[0]
    blk = 256
    return pl.pallas_call(
        _final_matmul_kernel,
        grid=(Bm // blk,),
        in_specs=[
            pl.BlockSpec((blk, K), lambda j: (j, 0)),
            pl.BlockSpec((Nm, K), lambda j: (0, 0)),
        ],
        out_specs=pl.BlockSpec((blk, Nm), lambda j: (j, 0)),
        out_shape=jax.ShapeDtypeStruct((Bm, Nm), F32),
    )(x, xe)


def _make_sc_layer(H, nch, cw, with_drug):
    mesh = plsc.VectorSubcoreMesh(core_axis_name="c", subcore_axis_name="s")
    hcn = H * nch
    rows_per_tile = (2 * NP) // 16
    nzchunks = rows_per_tile // 128

    def body(*refs):
        if with_drug:
            (src_hbm, dst_hbm, ssT_hbm, sdT_hbm, T_hbm, Z_hbm, Zd_hbm,
             didx_hbm, dfmix_hbm,
             ACC_hbm, DEN_hbm, XD_hbm,
             srcb, dstb, ssb, sdb, pcache,
             gbuf0, gbuf1, abuf0, abuf1, rowbuf0, rowbuf1, vch0, vch1,
             dibuf, dbuf, acc_sp, den_sp,
             semA0, semA1, semB0, semB1) = refs
        else:
            (src_hbm, dst_hbm, ssT_hbm, sdT_hbm, T_hbm, Z_hbm, Zd_hbm,
             ACC_hbm, DEN_hbm,
             srcb, dstb, ssb, sdb, pcache,
             gbuf0, gbuf1, abuf0, abuf1, rowbuf0, rowbuf1, vch0, vch1,
             acc_sp, den_sp,
             semA0, semA1, semB0, semB1) = refs

        cid = lax.axis_index("c")
        sid = lax.axis_index("s")
        wid = sid * 2 + cid
        tile_base = sid * EPT
        dslice = (2 * NP) // 16

        pltpu.sync_copy(src_hbm.at[pl.ds(sid * EPT, EPT)], srcb)
        pltpu.sync_copy(dst_hbm.at[pl.ds(sid * EPT, EPT)], dstb)

        if with_drug:
            pltpu.sync_copy(didx_hbm.at[pl.ds(wid * 32, 32)], dibuf)
            pltpu.async_copy(dfmix_hbm.at[dibuf], dbuf, semA0).wait()
            pltpu.sync_copy(dbuf, XD_hbm.at[pl.ds(wid * 32, 32)])

        def zero_acc():
            pltpu.sync_copy(
                Z_hbm, acc_sp.at[pl.ds(sid * rows_per_tile, rows_per_tile)])

        zero_acc()
        plsc.subcore_barrier()

        def head_body(h, _):
            pltpu.sync_copy(ssT_hbm.at[h], ssb)
            pltpu.sync_copy(sdT_hbm.at[h], sdb)

            den_core = h % 2

            @pl.when(cid == den_core)
            def _():
                pltpu.sync_copy(Zd_hbm.at[pl.ds(sid * dslice, dslice)],
                                den_sp.at[pl.ds(sid * dslice, dslice)])
            plsc.subcore_barrier()

            def cache_body(g, _):
                sv = srcb[pl.ds(g * 16, 16)]
                dv = dstb[pl.ds(g * 16, 16)]
                ss = plsc.load_gather(ssb, [sv])
                sd = plsc.load_gather(sdb, [dv])
                pos = (ss + sd) > 0
                bmul = jnp.where(pos, 0, 1)
                gid = tile_base + g * 16 + lax.iota(I32, 16)
                valid = gid < E
                grow = jnp.where(valid, bmul * NP + sv, NZ)
                arow = jnp.where(valid, bmul * NP + dv, NZ)
                pcache[pl.ds(g * 16, 16)] = grow | (arow << 15)
                return 0
            lax.fori_loop(0, EPT // 16, cache_body, 0)

            def dfill(j, abuf, vch):
                for off in range(8):
                    p = pcache[pl.ds(j * 128 + off * 16, 16)]
                    grow = p & 0x7FFF
                    bneg = (grow >= NP).astype(I32)
                    sv = grow - bneg * NP
                    ss = plsc.load_gather(ssb, [sv])
                    val = jnp.exp(jnp.where(bneg == 0, ss, 0.2 * ss))
                    val = jnp.where(sv == NZ, 0.0, val)
                    abuf[pl.ds(off * 16, 16)] = p >> 15
                    vch[pl.ds(off * 16, 16)] = val

            @pl.when(cid == den_core)
            def _():
                dfill(0, abuf0, vch0)
                pltpu.async_copy(vch0, den_sp.at[abuf0], semB0, add=True)
                dfill(1, abuf1, vch1)
                pltpu.async_copy(vch1, den_sp.at[abuf1], semB1, add=True)

                def dchunk(k, _):
                    j = 2 * k
                    pltpu.make_async_copy(vch0, den_sp.at[abuf0],
                                          semB0).wait()
                    dfill(j, abuf0, vch0)
                    pltpu.async_copy(vch0, den_sp.at[abuf0], semB0, add=True)
                    pltpu.make_async_copy(vch1, den_sp.at[abuf1],
                                          semB1).wait()
                    dfill(j + 1, abuf1, vch1)
                    pltpu.async_copy(vch1, den_sp.at[abuf1], semB1, add=True)
                    return 0
                lax.fori_loop(1, NJ // 2, dchunk, 0)

                pltpu.make_async_copy(vch0, den_sp.at[abuf0], semB0).wait()
                pltpu.make_async_copy(vch1, den_sp.at[abuf1], semB1).wait()
                plsc.subcore_barrier()
                pltpu.sync_copy(den_sp.at[pl.ds(sid * dslice, dslice)],
                                DEN_hbm.at[h, pl.ds(sid * dslice, dslice)])

            p = jnp.where(cid == den_core, 1, 0)
            ntrips = jnp.where(cid == den_core, nch // 2, (nch + 1) // 2)

            def ci_body(k, _):
                ci = jnp.minimum(p + 2 * k, nch - 1)
                base_t = (h * nch + ci) * 2 * NP

                def fill(j, gbuf, abuf):
                    for off in range(8):
                        p = pcache[pl.ds(j * 128 + off * 16, 16)]
                        gbuf[pl.ds(off * 16, 16)] = (p & 0x7FFF) + base_t
                        abuf[pl.ds(off * 16, 16)] = p >> 15

                fill(0, gbuf0, abuf0)
                pltpu.async_copy(T_hbm.at[gbuf0], rowbuf0, semA0)
                fill(1, gbuf1, abuf1)
                pltpu.async_copy(T_hbm.at[gbuf1], rowbuf1, semA1)
                pltpu.make_async_copy(T_hbm.at[gbuf0], rowbuf0, semA0).wait()
                pltpu.async_copy(rowbuf0, acc_sp.at[abuf0], semB0, add=True)
                pltpu.make_async_copy(T_hbm.at[gbuf1], rowbuf1, semA1).wait()
                pltpu.async_copy(rowbuf1, acc_sp.at[abuf1], semB1, add=True)

                def chunk(k2, _):
                    j = 2 * k2
                    pltpu.make_async_copy(rowbuf0, acc_sp.at[abuf0],
                                          semB0).wait()
                    fill(j, gbuf0, abuf0)
                    pltpu.async_copy(T_hbm.at[gbuf0], rowbuf0, semA0)
                    pltpu.make_async_copy(rowbuf1, acc_sp.at[abuf1],
                                          semB1).wait()
                    fill(j + 1, gbuf1, abuf1)
                    pltpu.async_copy(T_hbm.at[gbuf1], rowbuf1, semA1)
                    pltpu.make_async_copy(T_hbm.at[gbuf0], rowbuf0,
                                          semA0).wait()
                    pltpu.async_copy(rowbuf0, acc_sp.at[abuf0], semB0,
                                     add=True)
                    pltpu.make_async_copy(T_hbm.at[gbuf1], rowbuf1,
                                          semA1).wait()
                    pltpu.async_copy(rowbuf1, acc_sp.at[abuf1], semB1,
                                     add=True)
                    return 0
                lax.fori_loop(1, NJ // 2, chunk, 0)

                pltpu.make_async_copy(rowbuf0, acc_sp.at[abuf0],
                                      semB0).wait()
                pltpu.make_async_copy(rowbuf1, acc_sp.at[abuf1],
                                      semB1).wait()

                plsc.subcore_barrier()
                pltpu.sync_copy(
                    acc_sp.at[pl.ds(sid * rows_per_tile, rows_per_tile)],
                    ACC_hbm.at[pl.ds(base_t + sid * rows_per_tile,
                                     rows_per_tile)])
                plsc.subcore_barrier()
                zero_acc()
                plsc.subcore_barrier()
                return 0
            lax.fori_loop(0, ntrips, ci_body, 0)
            return 0

        lax.fori_loop(0, H, head_body, 0)

    out_type = [
        jax.ShapeDtypeStruct((hcn * 2 * NP, cw), F32),
        jax.ShapeDtypeStruct((H, 2 * NP), F32),
    ]
    if with_drug:
        out_type.append(jax.ShapeDtypeStruct((B, 128), F32))

    scratch = [
        pltpu.VMEM((EPT,), I32),
        pltpu.VMEM((EPT,), I32),
        pltpu.VMEM((NP,), F32),
        pltpu.VMEM((NP,), F32),
        pltpu.VMEM((EPT,), I32),
        pltpu.VMEM((128,), I32),
        pltpu.VMEM((128,), I32),
        pltpu.VMEM((128,), I32),
        pltpu.VMEM((128,), I32),
        pltpu.VMEM((128, cw), F32),
        pltpu.VMEM((128, cw), F32),
        pltpu.VMEM((128,), F32),
        pltpu.VMEM((128,), F32),
    ]
    if with_drug:
        scratch += [
            pltpu.VMEM((32,), I32),
            pltpu.VMEM((32, 128), F32),
        ]
    scratch += [
        pltpu.VMEM_SHARED((2 * NP, cw), F32),
        pltpu.VMEM_SHARED((2 * NP,), F32),
        pltpu.SemaphoreType.DMA,
        pltpu.SemaphoreType.DMA,
        pltpu.SemaphoreType.DMA,
        pltpu.SemaphoreType.DMA,
    ]

    return pl.kernel(body, out_type=out_type, mesh=mesh,
                     scratch_types=scratch,
                     compiler_params=pltpu.CompilerParams(
                         needs_layout_passes=False,
                         use_tc_tiling_on_sc=False),
                     name=f"sc_gat_h{H}_c{cw}")


def _block_diag_cols(a):
    H, oc = a.shape
    eye = jnp.eye(H, dtype=a.dtype)
    return (a[:, :, None] * eye[:, None, :]).reshape(H * oc, H)


def _gat_layer(xin, lg, lb, W, a_src, a_dst, bias, H, oc, nch, cw,
               src_pad, dst_pad, relu_flag, sc_fn, drug_args=None):
    AsrcM = _block_diag_cols(a_src)
    AdstM = _block_diag_cols(a_dst)
    xw, ssT, sdT = _gatA(xin, lg, lb, W, AsrcM, AdstM, H)
    xw4 = xw.reshape(NP, H, nch, cw)
    Tt = _gatB(xw4, ssT.reshape(H, 1, NP), H, nch, cw)
    Tflat = Tt.reshape(H * nch * 2 * NP, cw)
    Z = jnp.zeros(((2 * NP) // 16, cw), F32)
    Zd = jnp.zeros((2 * NP,), F32)
    if drug_args is not None:
        didx, dfmix = drug_args
        ACC, DEN, XD = sc_fn(src_pad, dst_pad, ssT, sdT, Tflat, Z, Zd,
                             didx, dfmix)
    else:
        ACC, DEN = sc_fn(src_pad, dst_pad, ssT, sdT, Tflat, Z, Zd)
        XD = None
    xout = _gatC(ACC, DEN, sdT, bias, H, nch, cw, relu_flag)
    return xout, XD


_SC1 = _make_sc_layer(10, 4, 32, True)
_SC2 = _make_sc_layer(10, 5, 40, False)
_SC3 = _make_sc_layer(1, 5, 40, False)


def kernel(index, x_e, edge_index_e, weight, df1, df2, fc_p1_W, fc_p1_b,
           fc1_W, fc1_b, fc2_W, fc2_b, n1_g, n1_b, ne1_g, ne1_b, ne2_g,
           ne2_b, ne3_g, ne3_b, W1, a_src1, a_dst1, b1, W2, a_src2, a_dst2,
           b2, W3, a_src3, a_dst3, b3, fcg4_W, fcg4_b):
    loops = jnp.arange(N, dtype=I32)
    src = jnp.concatenate([edge_index_e[0].astype(I32), loops])
    dst = jnp.concatenate([edge_index_e[1].astype(I32), loops])
    src_pad = jnp.pad(src, (0, EPAD - E))
    dst_pad = jnp.pad(dst, (0, EPAD - E))

    xe0 = jnp.pad(x_e, ((0, NP - N), (0, 0)))

    dfmix = _drug_mix(weight, df1, df2)
    didx = index.astype(I32)

    xe1, XD = _gat_layer(xe0, ne1_g, ne1_b, W1, a_src1, a_dst1, b1,
                         10, 128, 4, 32, src_pad, dst_pad, True, _SC1,
                         drug_args=(didx, dfmix))
    x = _drug_mlp(XD, fc_p1_W, fc_p1_b, n1_g, n1_b, fc1_W, fc1_b,
                  fc2_W, fc2_b)

    xe2, _ = _gat_layer(xe1, ne2_g, ne2_b, W2, a_src2, a_dst2, b2,
                        10, 200, 5, 40, src_pad, dst_pad, True, _SC2)
    xe3, _ = _gat_layer(xe2, ne3_g, ne3_b, W3, a_src3, a_dst3, b3,
                        1, 200, 5, 40, src_pad, dst_pad, False, _SC3)
    xe = _linear(xe3, fcg4_W, fcg4_b)[:N]

    xc = _final_matmul(x, xe)
    return (xc, x, xe)
